# Initial kernel scaffold; baseline (speedup 1.0000x reference)
import jax
import jax.numpy as jnp
from jax.experimental import pallas as pl


def kernel(x, edge_index, batch, y, params):
    raise NotImplementedError("write your pallas kernel here")


# speedup vs baseline: 3.5757x; 3.5757x over previous
import jax
import jax.numpy as jnp
from jax import lax
from jax.experimental import pallas as pl
from jax.experimental.pallas import tpu as pltpu
from jax.experimental.pallas import tpu_sc as plsc

N = 10000
E = 320000
D = 128
B = 64
NP = 10240
NC = 2
NS = 16
EPT = E // (NC * NS)
C = 80
RPT = NP // NS
_BN_EPS = 1e-5
_CP = pltpu.CompilerParams(vmem_limit_bytes=128 * 1024 * 1024)


def _split3(x):
    xh = x.astype(jnp.bfloat16)
    r = x - xh.astype(jnp.float32)
    xm = r.astype(jnp.bfloat16)
    xl = (r - xm.astype(jnp.float32)).astype(jnp.bfloat16)
    return xh, xm, xl


def _dot(p, q):
    return lax.dot_general(p, q, (((1,), (0,)), ((), ())),
                           preferred_element_type=jnp.float32)


def _dot6(x, y):
    xh, xm, xl = _split3(x)
    yh, ym, yl = _split3(y)
    return (_dot(xh, yh) + (_dot(xh, ym) + _dot(xm, yh))
            + (_dot(xh, yl) + _dot(xm, ym) + _dot(xl, yh)))


def _seg_sum_body(h_hbm, src_hbm, dst_hbm, zero_hbm, out_hbm,
                  acc, sidx, didx, rows, sem):
    c = lax.axis_index("c")
    s = lax.axis_index("s")
    pltpu.sync_copy(zero_hbm.at[pl.ds(s * RPT, RPT)],
                    acc.at[pl.ds(s * RPT, RPT)])
    plsc.subcore_barrier()
    base = (c * NS + s) * EPT

    def chunk(i, carry):
        off = base + i * C
        pltpu.sync_copy(src_hbm.at[pl.ds(off, C)], sidx)
        pltpu.sync_copy(dst_hbm.at[pl.ds(off, C)], didx)
        pltpu.async_copy(h_hbm.at[sidx], rows, sem).wait()
        pltpu.sync_copy(rows, acc.at[didx], add=True)
        return carry

    lax.fori_loop(0, EPT // C, chunk, 0)
    plsc.subcore_barrier()
    pltpu.sync_copy(acc.at[pl.ds(s * RPT, RPT)],
                    out_hbm.at[pl.ds(c * NP + s * RPT, RPT)])


def _seg_sum_sc(h, src, dst, zero):
    k = pl.kernel(
        _seg_sum_body,
        out_type=jax.ShapeDtypeStruct((NC * NP, D), jnp.float32),
        mesh=plsc.VectorSubcoreMesh(core_axis_name="c", subcore_axis_name="s",
                                    num_cores=NC, num_subcores=NS),
        scratch_types=[
            pltpu.VMEM_SHARED((NP, D), jnp.float32),
            pltpu.VMEM((C,), jnp.int32),
            pltpu.VMEM((C,), jnp.int32),
            pltpu.VMEM((C, D), jnp.float32),
            pltpu.SemaphoreType.DMA,
        ],
    )
    return k(h, src, dst, zero)


_CH = 1280
_NB = NP // _CH


def _mlp_body(pp_ref, h_ref, eps_ref, w1_ref, b1_ref, g1_ref, be1_ref,
              w2_ref, b2_ref, g_ref, be_ref, out_ref, z1s_ref):
    mask = (lax.broadcasted_iota(jnp.int32, (NP, 1), 0) < N).astype(jnp.float32)
    eps1 = 1.0 + eps_ref[0, 0]
    w1 = w1_ref[...]

    def c1(i, carry):
        sl = pl.ds(i * _CH, _CH)
        sl2 = pl.ds(NP + i * _CH, _CH)
        zc = eps1 * h_ref[sl, :] + (pp_ref[sl, :] + pp_ref[sl2, :])
        z1s_ref[sl, :] = _dot6(zc, w1) + b1_ref[...]
        return carry

    lax.fori_loop(0, _NB, c1, 0)
    z1 = z1s_ref[...] * mask
    m1 = jnp.sum(z1, axis=0, keepdims=True) / N
    d1 = (z1 - m1) * mask
    v1 = jnp.sum(d1 * d1, axis=0, keepdims=True) / N
    a1 = jnp.maximum(d1 / jnp.sqrt(v1 + _BN_EPS) * g1_ref[...] + be1_ref[...], 0.0)
    z1s_ref[...] = a1 * mask
    w2 = w2_ref[...]

    def c2(i, carry):
        sl = pl.ds(i * _CH, _CH)
        out_ref[sl, :] = _dot6(z1s_ref[sl, :], w2) + b2_ref[...]
        return carry

    lax.fori_loop(0, _NB, c2, 0)
    z2 = out_ref[...] * mask
    m2 = jnp.sum(z2, axis=0, keepdims=True) / N
    d2 = (z2 - m2) * mask
    v2 = jnp.sum(d2 * d2, axis=0, keepdims=True) / N
    a2 = jnp.maximum(d2 / jnp.sqrt(v2 + _BN_EPS) * g_ref[...] + be_ref[...], 0.0)
    out_ref[...] = a2 * mask


def _mlp_tc(pp, h, lp):
    return pl.pallas_call(
        _mlp_body,
        out_shape=jax.ShapeDtypeStruct((NP, D), jnp.float32),
        scratch_shapes=[pltpu.VMEM((NP, 2 * D), jnp.float32)],
        compiler_params=_CP,
    )(pp, h, lp['eps'].reshape(1, 1).astype(jnp.float32),
      lp['W1'], lp['b1'].reshape(1, 2 * D), lp['g1'].reshape(1, 2 * D),
      lp['be1'].reshape(1, 2 * D), lp['W2'], lp['b2'].reshape(1, D),
      lp['g'].reshape(1, D), lp['be'].reshape(1, D))


def _score_body(pp_ref, h_ref, wrel_ref, brel_ref, wroot_ref, out_ref):
    agg = pp_ref[:NP, :] + pp_ref[NP:, :]
    s = (_dot6(agg, wrel_ref[...]) + brel_ref[0, 0]
         + _dot6(h_ref[...], wroot_ref[...]))
    out_ref[...] = jnp.tanh(s)


def _score_tc(pp, h, sag):
    return pl.pallas_call(
        _score_body,
        out_shape=jax.ShapeDtypeStruct((NP, 1), jnp.float32),
        compiler_params=_CP,
    )(pp, h, sag['Wrel'], sag['brel'].reshape(1, 1), sag['Wroot'])


def _final_body(score_ref, score80_ref, batch_ref, batch80_ref, h_ref,
                w1_ref, b1_ref, g1_ref, be1_ref, w2_ref, b2_ref,
                emb_ref, logit_ref):
    score = score_ref[...]
    bcol = batch_ref[...]
    key = bcol.astype(jnp.float32) * 4.0 - score
    icol = lax.broadcasted_iota(jnp.int32, (NP, 1), 0)
    lane = lax.broadcasted_iota(jnp.int32, (1, 128), 1)

    def body(r, rank):
        kj = (batch80_ref[pl.ds(r, 1), :].astype(jnp.float32) * 4.0
              - score80_ref[pl.ds(r, 1), :])
        ij = r * 128 + lane
        lt = kj < key
        eqb = (kj == key) & (ij < icol)
        contrib = jnp.where(lt | eqb, 1.0, 0.0)
        return rank + jnp.sum(contrib, axis=1, keepdims=True)

    rank = lax.fori_loop(0, 80, body, jnp.zeros((NP, 1), jnp.float32))

    gid = lax.broadcasted_iota(jnp.int32, (1, B), 1)
    onehot = (bcol == gid).astype(jnp.float32)
    counts = jnp.sum(onehot, axis=0, keepdims=True)
    kper = jnp.floor((counts + 1.0) * 0.5)
    gj = lax.broadcasted_iota(jnp.int32, (B, 1), 0)
    lt_tri = (gj < gid).astype(jnp.float32)
    ch, cm, _ = _split3(counts)
    lt_b = lt_tri.astype(jnp.bfloat16)
    starts = _dot(ch, lt_b) + _dot(cm, lt_b)
    thr = starts + kper
    th, tm, _ = _split3(thr.reshape(B, 1))
    oh_b = onehot.astype(jnp.bfloat16)
    thr_col = _dot(oh_b, th) + _dot(oh_b, tm)
    sel = (rank < thr_col).astype(jnp.float32)
    xp = h_ref[...] * (score * sel)
    xh, xm, xl = _split3(xp)
    ct = lambda p, q: lax.dot_general(p, q, (((0,), (0,)), ((), ())),
                                     preferred_element_type=jnp.float32)
    emb = ct(oh_b, xh) + ct(oh_b, xm) + ct(oh_b, xl)
    emb_ref[...] = emb

    o = _dot6(emb, w1_ref[...]) + b1_ref[...]
    m = jnp.sum(o, axis=0, keepdims=True) / B
    d = o - m
    v = jnp.sum(d * d, axis=0, keepdims=True) / B
    o = jnp.maximum(d / jnp.sqrt(v + _BN_EPS) * g1_ref[...] + be1_ref[...], 0.0)
    logit_ref[...] = _dot6(o, w2_ref[...]) + b2_ref[...]


def _final_tc(score, score80, batch_col, batch80, h, params):
    return pl.pallas_call(
        _final_body,
        out_shape=[jax.ShapeDtypeStruct((B, D), jnp.float32),
                   jax.ShapeDtypeStruct((B, 64), jnp.float32)],
        compiler_params=_CP,
    )(score, score80, batch_col, batch80, h,
      params['lin1']['W'], params['lin1']['b'].reshape(1, D),
      params['bn1']['g'].reshape(1, D), params['bn1']['be'].reshape(1, D),
      params['lin2']['W'], params['lin2']['b'].reshape(1, 64))


def kernel(x, edge_index, batch, y, params):
    del y
    src = edge_index[0].astype(jnp.int32)
    dst = edge_index[1].astype(jnp.int32)
    zero = jnp.zeros((NP, D), jnp.float32)
    h = jnp.concatenate([x, jnp.zeros((NP - N, D), jnp.float32)], axis=0)
    for lp in params['layers']:
        pp = _seg_sum_sc(h, src, dst, zero)
        h = _mlp_tc(pp, h, lp)
    pp = _seg_sum_sc(h, src, dst, zero)
    score = _score_tc(pp, h, params['sag'])
    score80 = score.reshape(80, 128)
    bpad = jnp.concatenate([batch.astype(jnp.int32),
                            jnp.full((NP - N,), B, jnp.int32)])
    emb, logits = _final_tc(score, score80, bpad.reshape(NP, 1),
                            bpad.reshape(80, 128), h, params)
    return emb, logits
